# baseline (speedup 1.0000x reference)
import math

import jax
import jax.numpy as jnp
from jax.experimental import pallas as pl
from jax.experimental.pallas import tpu as pltpu

B = 16
Z = 4096
D = 768
K = 32
NZ = 4
ZC = Z // NZ
SCALE = 1.0 / math.sqrt(D)


def _fused_kernel(txt_ref, zone_ref, wt_ref, bt_ref, wz_ref, bz_ref,
                  mask_ref, out_ref, acc_ref):
    b = pl.program_id(0)
    zc = pl.program_id(1)
    txt_b = txt_ref[pl.ds(b, 1), :]
    t = jnp.tanh(
        jax.lax.dot_general(txt_b, wt_ref[...], (((1,), (1,)), ((), ())),
                            preferred_element_type=jnp.float32)
        + bt_ref[...]
    ) * SCALE
    z = jax.lax.dot_general(wz_ref[...], zone_ref[0],
                            (((1,), (1,)), ((), ())),
                            preferred_element_type=jnp.float32)
    z = jnp.tanh(z + bz_ref[...])
    x = jax.lax.dot_general(t, z, (((1,), (0,)), ((), ())),
                            preferred_element_type=jnp.float32)
    e = jnp.where(mask_ref[0] != 0, 0.0, jnp.exp(x))
    total = jnp.where(zc == 0, 0.0, acc_ref[0, 0]) + jnp.sum(e)
    acc_ref[0, 0] = total
    out_ref[0, :, pl.ds(zc * ZC, ZC)] = e

    @pl.when(zc == NZ - 1)
    def _normalize():
        out_ref[0] = out_ref[0] * (1.0 / total)


def kernel(txt_embeds, zone_embeds, W_txt, b_txt, W_zone, b_zone, mask):
    out = pl.pallas_call(
        _fused_kernel,
        grid=(B, NZ),
        in_specs=[
            pl.BlockSpec((B, D), lambda b, zc: (0, 0)),
            pl.BlockSpec((1, ZC, D), lambda b, zc: (b, zc, 0)),
            pl.BlockSpec((K, D), lambda b, zc: (0, 0)),
            pl.BlockSpec((1, K), lambda b, zc: (0, 0)),
            pl.BlockSpec((K, D), lambda b, zc: (0, 0)),
            pl.BlockSpec((K, 1), lambda b, zc: (0, 0)),
            pl.BlockSpec((1, 1, ZC), lambda b, zc: (b, 0, zc)),
        ],
        out_specs=pl.BlockSpec((1, 1, Z), lambda b, zc: (b, 0, 0)),
        out_shape=jax.ShapeDtypeStruct((B, 1, Z), jnp.float32),
        scratch_shapes=[pltpu.SMEM((1, 1), jnp.float32)],
        compiler_params=pltpu.CompilerParams(
            dimension_semantics=("parallel", "arbitrary")),
    )(txt_embeds, zone_embeds, W_txt, b_txt.reshape(1, K),
      W_zone, b_zone.reshape(K, 1), mask.astype(jnp.int32).reshape(B, 1, Z))
    return out.reshape(B, Z)


# speedup vs baseline: 1.2535x; 1.2535x over previous
import math

import jax
import jax.numpy as jnp
from jax.experimental import pallas as pl
from jax.experimental.pallas import tpu as pltpu

B = 16
Z = 4096
D = 768
K = 32
NZ = 2
ZC = Z // NZ
SCALE = 1.0 / math.sqrt(D)


def _fused_kernel(txt_ref, zone_ref, wt_ref, bt_ref, wz_ref, bz_ref,
                  mask_ref, out_ref, t_ref, acc_ref):
    b = pl.program_id(0)
    zc = pl.program_id(1)

    @pl.when(zc == 0)
    def _project_txt():
        txt_b = txt_ref[pl.ds(b, 1), :]
        t_ref[...] = jnp.tanh(
            jax.lax.dot_general(txt_b, wt_ref[...], (((1,), (1,)), ((), ())),
                                preferred_element_type=jnp.float32)
            + bt_ref[...]
        ) * SCALE

    z = jax.lax.dot_general(wz_ref[...], zone_ref[0],
                            (((1,), (1,)), ((), ())),
                            preferred_element_type=jnp.float32)
    z = jnp.tanh(z + bz_ref[...])
    x = jax.lax.dot_general(t_ref[...], z, (((1,), (0,)), ((), ())),
                            preferred_element_type=jnp.float32)
    e = jnp.where(mask_ref[0] != 0, 0.0, jnp.exp(x))
    total = jnp.where(zc == 0, 0.0, acc_ref[0, 0]) + jnp.sum(e)
    acc_ref[0, 0] = total
    out_ref[0, :, pl.ds(zc * ZC, ZC)] = e

    @pl.when(zc == NZ - 1)
    def _normalize():
        out_ref[0] = out_ref[0] * (1.0 / total)


def kernel(txt_embeds, zone_embeds, W_txt, b_txt, W_zone, b_zone, mask):
    out = pl.pallas_call(
        _fused_kernel,
        grid=(B, NZ),
        in_specs=[
            pl.BlockSpec((B, D), lambda b, zc: (0, 0)),
            pl.BlockSpec((1, ZC, D), lambda b, zc: (b, zc, 0)),
            pl.BlockSpec((K, D), lambda b, zc: (0, 0)),
            pl.BlockSpec((1, K), lambda b, zc: (0, 0)),
            pl.BlockSpec((K, D), lambda b, zc: (0, 0)),
            pl.BlockSpec((K, 1), lambda b, zc: (0, 0)),
            pl.BlockSpec((1, 1, ZC), lambda b, zc: (b, 0, zc)),
        ],
        out_specs=pl.BlockSpec((1, 1, Z), lambda b, zc: (b, 0, 0)),
        out_shape=jax.ShapeDtypeStruct((B, 1, Z), jnp.float32),
        scratch_shapes=[pltpu.VMEM((1, K), jnp.float32),
                        pltpu.SMEM((1, 1), jnp.float32)],
        compiler_params=pltpu.CompilerParams(
            dimension_semantics=("parallel", "arbitrary")),
    )(txt_embeds, zone_embeds, W_txt, b_txt.reshape(1, K),
      W_zone, b_zone.reshape(K, 1), mask.astype(jnp.int32).reshape(B, 1, Z))
    return out.reshape(B, Z)


# speedup vs baseline: 1.4541x; 1.1600x over previous
import math

import jax
import jax.numpy as jnp
from jax.experimental import pallas as pl
from jax.experimental.pallas import tpu as pltpu

B = 16
Z = 4096
D = 768
K = 32
SCALE = 1.0 / math.sqrt(D)


def _fused_kernel(txt_ref, zone_ref, wt_ref, bt_ref, wz_ref, bz_ref,
                  mask_ref, out_ref):
    b = pl.program_id(0)
    txt_b = txt_ref[pl.ds(b, 1), :]
    t = jnp.tanh(
        jax.lax.dot_general(txt_b, wt_ref[...], (((1,), (1,)), ((), ())),
                            preferred_element_type=jnp.float32)
        + bt_ref[...]
    ) * SCALE
    z = jax.lax.dot_general(wz_ref[...], zone_ref[0],
                            (((1,), (1,)), ((), ())),
                            preferred_element_type=jnp.float32)
    z = jnp.tanh(z + bz_ref[...])
    x = jax.lax.dot_general(t, z, (((1,), (0,)), ((), ())),
                            preferred_element_type=jnp.float32)
    e = jnp.where(mask_ref[0] != 0, 0.0, jnp.exp(x))
    out_ref[0] = e * (1.0 / jnp.sum(e))


def kernel(txt_embeds, zone_embeds, W_txt, b_txt, W_zone, b_zone, mask):
    out = pl.pallas_call(
        _fused_kernel,
        grid=(B,),
        in_specs=[
            pl.BlockSpec((B, D), lambda b: (0, 0)),
            pl.BlockSpec((1, Z, D), lambda b: (b, 0, 0)),
            pl.BlockSpec((K, D), lambda b: (0, 0)),
            pl.BlockSpec((1, K), lambda b: (0, 0)),
            pl.BlockSpec((K, D), lambda b: (0, 0)),
            pl.BlockSpec((K, 1), lambda b: (0, 0)),
            pl.BlockSpec((1, 1, Z), lambda b: (b, 0, 0)),
        ],
        out_specs=pl.BlockSpec((1, 1, Z), lambda b: (b, 0, 0)),
        out_shape=jax.ShapeDtypeStruct((B, 1, Z), jnp.float32),
        compiler_params=pltpu.CompilerParams(
            dimension_semantics=("parallel",)),
    )(txt_embeds, zone_embeds, W_txt, b_txt.reshape(1, K),
      W_zone, b_zone.reshape(K, 1), mask.astype(jnp.int32).reshape(B, 1, Z))
    return out.reshape(B, Z)
